# baseline (speedup 1.0000x reference)
import functools

import jax
import jax.numpy as jnp
from jax import lax
from jax.experimental import pallas as pl
from jax.experimental.pallas import tpu as pltpu
from jax.experimental.pallas import tpu_sc as plsc

NC = 2
NS = 16
LANES = 16
CHUNK = 80
HALF = 5120
NPAD = NC * HALF
TRASH = 128
ACC_ROWS = HALF + TRASH
DW = 128


def _sc_mesh():
    return plsc.VectorSubcoreMesh(
        core_axis_name="c", subcore_axis_name="s", num_cores=NC, num_subcores=NS
    )


def _redirect_body(s_ref, d_ref, os_ref, od_ref):
    s = s_ref[...]
    d = d_ref[...]
    for c in range(NC):
        base = c * HALF
        sl = s - base
        dl = d - base
        os_ref[c] = jnp.where((sl >= 0) & (sl < HALF), sl, HALF)
        od_ref[c] = jnp.where((dl >= 0) & (dl < HALF), dl, HALF)


def _tc_redirect(src2, dst2):
    r, w = src2.shape
    oshape = jax.ShapeDtypeStruct((NC, r, w), jnp.int32)
    return pl.pallas_call(
        _redirect_body,
        out_shape=[oshape, oshape],
    )(src2, dst2)


def _make_degree_kernel(nch):
    out_stripe = HALF // NS
    acc_stripe = ACC_ROWS // NS

    @functools.partial(
        pl.kernel,
        out_type=[
            jax.ShapeDtypeStruct((NC, HALF, DW), jnp.float32),
            jax.ShapeDtypeStruct((NC, HALF, DW), jnp.float32),
        ],
        mesh=_sc_mesh(),
        scratch_types=[
            pltpu.VMEM((nch, CHUNK), jnp.int32),
            pltpu.VMEM((nch, CHUNK), jnp.int32),
            pltpu.VMEM((CHUNK, DW), jnp.float32),
            pltpu.VMEM_SHARED((ACC_ROWS, DW), jnp.float32),
        ],
    )
    def degree_kernel(srcr_hbm, dstr_hbm, zeros_hbm, ones_hbm, osrc_hbm,
                      odst_hbm, srcr_v, dstr_v, ones_v, acc_sh):
        c = lax.axis_index("c")
        s = lax.axis_index("s")
        wid = c * NS + s

        pltpu.sync_copy(srcr_hbm.at[wid], srcr_v)
        pltpu.sync_copy(dstr_hbm.at[wid], dstr_v)
        pltpu.sync_copy(ones_hbm, ones_v)

        zbase = s * acc_stripe
        base = s * out_stripe

        def one_pass(idx_v, out_hbm):
            pltpu.sync_copy(zeros_hbm.at[pl.ds(zbase, acc_stripe)],
                            acc_sh.at[pl.ds(zbase, acc_stripe)])
            plsc.subcore_barrier()

            def body(j, carry):
                pltpu.sync_copy(ones_v, acc_sh.at[idx_v.at[j]], add=True)
                return carry

            lax.fori_loop(0, nch, body, 0)
            plsc.subcore_barrier()
            pltpu.sync_copy(acc_sh.at[pl.ds(base, out_stripe)],
                            out_hbm.at[c, pl.ds(base, out_stripe)])
            plsc.subcore_barrier()

        one_pass(srcr_v, osrc_hbm)
        one_pass(dstr_v, odst_hbm)

    return degree_kernel


def _make_agg_kernel(d, nch):
    out_stripe = HALF // NS
    acc_stripe = ACC_ROWS // NS

    @functools.partial(
        pl.kernel,
        out_type=jax.ShapeDtypeStruct((NC, HALF, d), jnp.float32),
        mesh=_sc_mesh(),
        scratch_types=[
            pltpu.VMEM((nch + 2, CHUNK), jnp.int32),
            pltpu.VMEM((nch, CHUNK), jnp.int32),
            pltpu.VMEM((CHUNK, d), jnp.float32),
            pltpu.VMEM((CHUNK, d), jnp.float32),
            pltpu.VMEM_SHARED((ACC_ROWS, d), jnp.float32),
            pltpu.SemaphoreType.DMA,
            pltpu.SemaphoreType.DMA,
        ],
    )
    def agg_kernel(x_hbm, src_hbm, dstr_hbm, zeros_hbm, out_hbm,
                   src_v, dstr_v, rows0_v, rows1_v, acc_sh, sem0, sem1):
        c = lax.axis_index("c")
        s = lax.axis_index("s")
        wid = c * NS + s

        pltpu.sync_copy(src_hbm.at[s], src_v)
        pltpu.sync_copy(dstr_hbm.at[wid], dstr_v)

        zbase = s * acc_stripe
        pltpu.sync_copy(zeros_hbm.at[pl.ds(zbase, acc_stripe)],
                        acc_sh.at[pl.ds(zbase, acc_stripe)])

        plsc.subcore_barrier()

        pltpu.async_copy(x_hbm.at[src_v.at[0]], rows0_v, sem0)
        pltpu.async_copy(x_hbm.at[src_v.at[1]], rows1_v, sem1)

        def body(k, carry):
            j0 = 2 * k
            j1 = 2 * k + 1
            pltpu.make_async_copy(x_hbm.at[src_v.at[j0]], rows0_v, sem0).wait()
            pltpu.sync_copy(rows0_v, acc_sh.at[dstr_v.at[j0]], add=True)
            pltpu.async_copy(x_hbm.at[src_v.at[j0 + 2]], rows0_v, sem0)
            pltpu.make_async_copy(x_hbm.at[src_v.at[j1]], rows1_v, sem1).wait()
            pltpu.sync_copy(rows1_v, acc_sh.at[dstr_v.at[j1]], add=True)
            pltpu.async_copy(x_hbm.at[src_v.at[j1 + 2]], rows1_v, sem1)
            return carry

        lax.fori_loop(0, nch // 2, body, 0)

        pltpu.make_async_copy(x_hbm.at[src_v.at[0]], rows0_v, sem0).wait()
        pltpu.make_async_copy(x_hbm.at[src_v.at[1]], rows1_v, sem1).wait()

        plsc.subcore_barrier()

        base = s * out_stripe
        pltpu.sync_copy(acc_sh.at[pl.ds(base, out_stripe)],
                        out_hbm.at[c, pl.ds(base, out_stripe)])

    return agg_kernel


def _norm_body(dps_ref, dpd_ref, ns_ref, nd_ref):
    deg_s = dps_ref[:, 0:1]
    deg_d = dpd_ref[:, 0:1]
    ns_ref[...] = jnp.where(deg_s > 0, lax.rsqrt(jnp.maximum(deg_s, 1.0)), 0.0)
    nd_ref[...] = jnp.where(deg_d > 0, lax.rsqrt(jnp.maximum(deg_d, 1.0)), 0.0)


def _tc_norms(deg_src, deg_dst, block_rows):
    n = deg_src.shape[0]
    grid = n // block_rows
    return pl.pallas_call(
        _norm_body,
        grid=(grid,),
        in_specs=[
            pl.BlockSpec((block_rows, DW), lambda i: (i, 0)),
            pl.BlockSpec((block_rows, DW), lambda i: (i, 0)),
        ],
        out_specs=[
            pl.BlockSpec((block_rows, 1), lambda i: (i, 0)),
            pl.BlockSpec((block_rows, 1), lambda i: (i, 0)),
        ],
        out_shape=[
            jax.ShapeDtypeStruct((n, 1), jnp.float32),
            jax.ShapeDtypeStruct((n, 1), jnp.float32),
        ],
    )(deg_src, deg_dst)


def _scale_body(x_ref, n_ref, o_ref):
    o_ref[...] = x_ref[...] * n_ref[...]


def _tc_scale(x, nvec, block_rows):
    n, d = x.shape
    grid = n // block_rows
    return pl.pallas_call(
        _scale_body,
        grid=(grid,),
        in_specs=[
            pl.BlockSpec((block_rows, d), lambda i: (i, 0)),
            pl.BlockSpec((block_rows, 1), lambda i: (i, 0)),
        ],
        out_specs=pl.BlockSpec((block_rows, d), lambda i: (i, 0)),
        out_shape=jax.ShapeDtypeStruct((n, d), jnp.float32),
    )(x, nvec)


def _layer_body(p_ref, nd_ref, ps_ref, w_ref, b_ref, o_ref):
    agg = p_ref[...] * nd_ref[...]
    y = jnp.dot(agg, w_ref[...], preferred_element_type=jnp.float32) + b_ref[...]
    o_ref[...] = jnp.maximum(y, 0.0) * ps_ref[...]


def _tc_layer(p, norm_dst, post_scale, w, b, block_rows):
    n, d = p.shape
    grid = n // block_rows
    return pl.pallas_call(
        _layer_body,
        grid=(grid,),
        in_specs=[
            pl.BlockSpec((block_rows, d), lambda i: (i, 0)),
            pl.BlockSpec((block_rows, 1), lambda i: (i, 0)),
            pl.BlockSpec((block_rows, 1), lambda i: (i, 0)),
            pl.BlockSpec((d, d), lambda i: (0, 0)),
            pl.BlockSpec((1, d), lambda i: (0, 0)),
        ],
        out_specs=pl.BlockSpec((block_rows, d), lambda i: (i, 0)),
        out_shape=jax.ShapeDtypeStruct((n, d), jnp.float32),
    )(p, norm_dst, post_scale, w, b)


def kernel(features, edge_index, W1, b1, W2, b2):
    n, d = features.shape
    e = edge_index.shape[1]
    assert n <= NPAD and d % 128 == 0
    assert e % (NS * CHUNK) == 0
    nch = e // (NS * CHUNK)

    src = edge_index[0]
    dst = edge_index[1]
    src2 = src.reshape(e // 128, 128)
    dst2 = dst.reshape(e // 128, 128)
    srcr, dstr = _tc_redirect(src2, dst2)
    srcr_c = srcr.reshape(NC * NS, nch, CHUNK)
    dstr_c = dstr.reshape(NC * NS, nch, CHUNK)
    src_c = jnp.pad(src.reshape(NS, nch, CHUNK), ((0, 0), (0, 2), (0, 0)))

    zeros_deg = jnp.zeros((ACC_ROWS, DW), jnp.float32)
    zeros_agg = jnp.zeros((ACC_ROWS, d), jnp.float32)
    ones_deg = jnp.ones((CHUNK, DW), jnp.float32)

    block_rows = 2048
    assert NPAD % block_rows == 0
    degp_src, degp_dst = _make_degree_kernel(nch)(
        srcr_c, dstr_c, zeros_deg, ones_deg)
    norm_src, norm_dst = _tc_norms(
        degp_src.reshape(NPAD, DW), degp_dst.reshape(NPAD, DW), block_rows
    )

    features_p = jnp.pad(features, ((0, NPAD - n), (0, 0)))
    agg = _make_agg_kernel(d, nch)

    x0 = _tc_scale(features_p, norm_src, block_rows)
    p = agg(x0, src_c, dstr_c, zeros_agg)
    h1 = _tc_layer(p.reshape(NPAD, d), norm_dst, norm_src, W1,
                   b1.reshape(1, d), block_rows)
    p2 = agg(h1, src_c, dstr_c, zeros_agg)
    ones = jnp.ones((NPAD, 1), jnp.float32)
    h2 = _tc_layer(p2.reshape(NPAD, d), norm_dst, ones, W2,
                   b2.reshape(1, d), block_rows)
    return h2[:n]


# speedup vs baseline: 1.0218x; 1.0218x over previous
import functools

import jax
import jax.numpy as jnp
from jax import lax
from jax.experimental import pallas as pl
from jax.experimental.pallas import tpu as pltpu
from jax.experimental.pallas import tpu_sc as plsc

NC = 2
NS = 16
LANES = 16
CHUNK = 128
HALF = 5120
NPAD = NC * HALF
TRASH = 128
ACC_ROWS = HALF + TRASH
DW = 128


def _sc_mesh():
    return plsc.VectorSubcoreMesh(
        core_axis_name="c", subcore_axis_name="s", num_cores=NC, num_subcores=NS
    )


def _redirect_body(s_ref, d_ref, os_ref, od_ref):
    s = s_ref[...]
    d = d_ref[...]
    for c in range(NC):
        base = c * HALF
        sl = s - base
        dl = d - base
        os_ref[c] = jnp.where((sl >= 0) & (sl < HALF), sl, HALF)
        od_ref[c] = jnp.where((dl >= 0) & (dl < HALF), dl, HALF)


def _tc_redirect(src2, dst2):
    r, w = src2.shape
    oshape = jax.ShapeDtypeStruct((NC, r, w), jnp.int32)
    return pl.pallas_call(
        _redirect_body,
        out_shape=[oshape, oshape],
    )(src2, dst2)


def _make_degree_kernel(nch):
    out_stripe = HALF // NS
    acc_stripe = ACC_ROWS // NS

    @functools.partial(
        pl.kernel,
        out_type=[
            jax.ShapeDtypeStruct((NC, HALF, DW), jnp.float32),
            jax.ShapeDtypeStruct((NC, HALF, DW), jnp.float32),
        ],
        mesh=_sc_mesh(),
        scratch_types=[
            pltpu.VMEM((nch, CHUNK), jnp.int32),
            pltpu.VMEM((nch, CHUNK), jnp.int32),
            pltpu.VMEM((CHUNK, DW), jnp.float32),
            pltpu.VMEM_SHARED((ACC_ROWS, DW), jnp.float32),
        ],
    )
    def degree_kernel(srcr_hbm, dstr_hbm, zeros_hbm, ones_hbm, osrc_hbm,
                      odst_hbm, srcr_v, dstr_v, ones_v, acc_sh):
        c = lax.axis_index("c")
        s = lax.axis_index("s")
        wid = c * NS + s

        pltpu.sync_copy(srcr_hbm.at[wid], srcr_v)
        pltpu.sync_copy(dstr_hbm.at[wid], dstr_v)
        pltpu.sync_copy(ones_hbm, ones_v)

        zbase = s * acc_stripe
        base = s * out_stripe

        def one_pass(idx_v, out_hbm):
            pltpu.sync_copy(zeros_hbm.at[pl.ds(zbase, acc_stripe)],
                            acc_sh.at[pl.ds(zbase, acc_stripe)])
            plsc.subcore_barrier()

            def body(j, carry):
                pltpu.sync_copy(ones_v, acc_sh.at[idx_v.at[j]], add=True)
                return carry

            lax.fori_loop(0, nch, body, 0)
            plsc.subcore_barrier()
            pltpu.sync_copy(acc_sh.at[pl.ds(base, out_stripe)],
                            out_hbm.at[c, pl.ds(base, out_stripe)])
            plsc.subcore_barrier()

        one_pass(srcr_v, osrc_hbm)
        one_pass(dstr_v, odst_hbm)

    return degree_kernel


def _make_agg_kernel(d, nch):
    out_stripe = HALF // NS
    acc_stripe = ACC_ROWS // NS

    @functools.partial(
        pl.kernel,
        out_type=jax.ShapeDtypeStruct((NC, HALF, d), jnp.float32),
        mesh=_sc_mesh(),
        scratch_types=[
            pltpu.VMEM((nch, CHUNK), jnp.int32),
            pltpu.VMEM((nch, CHUNK), jnp.int32),
            pltpu.VMEM((CHUNK, d), jnp.float32),
            pltpu.VMEM((CHUNK, d), jnp.float32),
            pltpu.VMEM_SHARED((ACC_ROWS, d), jnp.float32),
            pltpu.SemaphoreType.DMA,
            pltpu.SemaphoreType.DMA,
        ],
    )
    def agg_kernel(x_hbm, src_hbm, dstr_hbm, zeros_hbm, out_hbm,
                   src_v, dstr_v, rows0_v, rows1_v, acc_sh, sem0, sem1):
        c = lax.axis_index("c")
        s = lax.axis_index("s")
        wid = c * NS + s

        pltpu.sync_copy(src_hbm.at[s], src_v)
        pltpu.sync_copy(dstr_hbm.at[wid], dstr_v)

        zbase = s * acc_stripe
        pltpu.sync_copy(zeros_hbm.at[pl.ds(zbase, acc_stripe)],
                        acc_sh.at[pl.ds(zbase, acc_stripe)])

        plsc.subcore_barrier()

        def body(j, carry):
            pltpu.async_copy(x_hbm.at[src_v.at[j]], rows0_v, sem0).wait()
            pltpu.sync_copy(rows0_v, acc_sh.at[dstr_v.at[j]], add=True)
            return carry

        lax.fori_loop(0, nch, body, 0)

        plsc.subcore_barrier()

        base = s * out_stripe
        pltpu.sync_copy(acc_sh.at[pl.ds(base, out_stripe)],
                        out_hbm.at[c, pl.ds(base, out_stripe)])

    return agg_kernel


def _norm_body(dps_ref, dpd_ref, ns_ref, nd_ref):
    deg_s = dps_ref[:, 0:1]
    deg_d = dpd_ref[:, 0:1]
    ns_ref[...] = jnp.where(deg_s > 0, lax.rsqrt(jnp.maximum(deg_s, 1.0)), 0.0)
    nd_ref[...] = jnp.where(deg_d > 0, lax.rsqrt(jnp.maximum(deg_d, 1.0)), 0.0)


def _tc_norms(deg_src, deg_dst, block_rows):
    n = deg_src.shape[0]
    grid = n // block_rows
    return pl.pallas_call(
        _norm_body,
        grid=(grid,),
        in_specs=[
            pl.BlockSpec((block_rows, DW), lambda i: (i, 0)),
            pl.BlockSpec((block_rows, DW), lambda i: (i, 0)),
        ],
        out_specs=[
            pl.BlockSpec((block_rows, 1), lambda i: (i, 0)),
            pl.BlockSpec((block_rows, 1), lambda i: (i, 0)),
        ],
        out_shape=[
            jax.ShapeDtypeStruct((n, 1), jnp.float32),
            jax.ShapeDtypeStruct((n, 1), jnp.float32),
        ],
    )(deg_src, deg_dst)


def _scale_body(x_ref, n_ref, o_ref):
    o_ref[...] = x_ref[...] * n_ref[...]


def _tc_scale(x, nvec, block_rows):
    n, d = x.shape
    grid = n // block_rows
    return pl.pallas_call(
        _scale_body,
        grid=(grid,),
        in_specs=[
            pl.BlockSpec((block_rows, d), lambda i: (i, 0)),
            pl.BlockSpec((block_rows, 1), lambda i: (i, 0)),
        ],
        out_specs=pl.BlockSpec((block_rows, d), lambda i: (i, 0)),
        out_shape=jax.ShapeDtypeStruct((n, d), jnp.float32),
    )(x, nvec)


def _layer_body(p_ref, nd_ref, ps_ref, w_ref, b_ref, o_ref):
    agg = p_ref[...] * nd_ref[...]
    y = jnp.dot(agg, w_ref[...], preferred_element_type=jnp.float32) + b_ref[...]
    o_ref[...] = jnp.maximum(y, 0.0) * ps_ref[...]


def _tc_layer(p, norm_dst, post_scale, w, b, block_rows):
    n, d = p.shape
    grid = n // block_rows
    return pl.pallas_call(
        _layer_body,
        grid=(grid,),
        in_specs=[
            pl.BlockSpec((block_rows, d), lambda i: (i, 0)),
            pl.BlockSpec((block_rows, 1), lambda i: (i, 0)),
            pl.BlockSpec((block_rows, 1), lambda i: (i, 0)),
            pl.BlockSpec((d, d), lambda i: (0, 0)),
            pl.BlockSpec((1, d), lambda i: (0, 0)),
        ],
        out_specs=pl.BlockSpec((block_rows, d), lambda i: (i, 0)),
        out_shape=jax.ShapeDtypeStruct((n, d), jnp.float32),
    )(p, norm_dst, post_scale, w, b)


def kernel(features, edge_index, W1, b1, W2, b2):
    n, d = features.shape
    e = edge_index.shape[1]
    assert n <= NPAD and d % 128 == 0
    nch = -(-e // (NS * CHUNK))

    ep = NS * nch * CHUNK
    src_deg = jnp.pad(edge_index[0], (0, ep - e), constant_values=NPAD)
    dst = jnp.pad(edge_index[1], (0, ep - e), constant_values=NPAD)
    src2 = src_deg.reshape(ep // 128, 128)
    dst2 = dst.reshape(ep // 128, 128)
    srcr, dstr = _tc_redirect(src2, dst2)
    srcr_c = srcr.reshape(NC * NS, nch, CHUNK)
    dstr_c = dstr.reshape(NC * NS, nch, CHUNK)
    src_c = jnp.pad(edge_index[0], (0, ep - e)).reshape(NS, nch, CHUNK)

    zeros_deg = jnp.zeros((ACC_ROWS, DW), jnp.float32)
    zeros_agg = jnp.zeros((ACC_ROWS, d), jnp.float32)
    ones_deg = jnp.ones((CHUNK, DW), jnp.float32)

    block_rows = 2048
    assert NPAD % block_rows == 0
    degp_src, degp_dst = _make_degree_kernel(nch)(
        srcr_c, dstr_c, zeros_deg, ones_deg)
    norm_src, norm_dst = _tc_norms(
        degp_src.reshape(NPAD, DW), degp_dst.reshape(NPAD, DW), block_rows
    )

    features_p = jnp.pad(features, ((0, NPAD - n), (0, 0)))
    agg = _make_agg_kernel(d, nch)

    x0 = _tc_scale(features_p, norm_src, block_rows)
    p = agg(x0, src_c, dstr_c, zeros_agg)
    h1 = _tc_layer(p.reshape(NPAD, d), norm_dst, norm_src, W1,
                   b1.reshape(1, d), block_rows)
    p2 = agg(h1, src_c, dstr_c, zeros_agg)
    ones = jnp.ones((NPAD, 1), jnp.float32)
    h2 = _tc_layer(p2.reshape(NPAD, d), norm_dst, ones, W2,
                   b2.reshape(1, d), block_rows)
    return h2[:n]


# speedup vs baseline: 1.0725x; 1.0496x over previous
import functools

import jax
import jax.numpy as jnp
from jax import lax
from jax.experimental import pallas as pl
from jax.experimental.pallas import tpu as pltpu
from jax.experimental.pallas import tpu_sc as plsc

NC = 2
NS = 16
LANES = 16
CHUNK = 80
HALF = 5120
NPAD = NC * HALF
TRASH = 128
ACC_ROWS = HALF + TRASH
DW = 128


def _sc_mesh():
    return plsc.VectorSubcoreMesh(
        core_axis_name="c", subcore_axis_name="s", num_cores=NC, num_subcores=NS
    )


def _redirect_body(s_ref, d_ref, os_ref, od_ref):
    s = s_ref[...]
    d = d_ref[...]
    for c in range(NC):
        base = c * HALF
        sl = s - base
        dl = d - base
        os_ref[c] = jnp.where((sl >= 0) & (sl < HALF), sl, HALF)
        od_ref[c] = jnp.where((dl >= 0) & (dl < HALF), dl, HALF)


def _tc_redirect(src2, dst2):
    r, w = src2.shape
    oshape = jax.ShapeDtypeStruct((NC, r, w), jnp.int32)
    return pl.pallas_call(
        _redirect_body,
        out_shape=[oshape, oshape],
    )(src2, dst2)


def _make_degree_kernel(nch):
    out_stripe = HALF // NS
    acc_stripe = ACC_ROWS // NS

    @functools.partial(
        pl.kernel,
        out_type=[
            jax.ShapeDtypeStruct((NC, HALF, DW), jnp.float32),
            jax.ShapeDtypeStruct((NC, HALF, DW), jnp.float32),
        ],
        mesh=_sc_mesh(),
        scratch_types=[
            pltpu.VMEM((nch, CHUNK), jnp.int32),
            pltpu.VMEM((nch, CHUNK), jnp.int32),
            pltpu.VMEM((CHUNK, DW), jnp.float32),
            pltpu.VMEM_SHARED((ACC_ROWS, DW), jnp.float32),
        ],
    )
    def degree_kernel(srcr_hbm, dstr_hbm, zeros_hbm, ones_hbm, osrc_hbm,
                      odst_hbm, srcr_v, dstr_v, ones_v, acc_sh):
        c = lax.axis_index("c")
        s = lax.axis_index("s")
        wid = c * NS + s

        pltpu.sync_copy(srcr_hbm.at[wid], srcr_v)
        pltpu.sync_copy(dstr_hbm.at[wid], dstr_v)
        pltpu.sync_copy(ones_hbm, ones_v)

        zbase = s * acc_stripe
        base = s * out_stripe

        def one_pass(idx_v, out_hbm):
            pltpu.sync_copy(zeros_hbm.at[pl.ds(zbase, acc_stripe)],
                            acc_sh.at[pl.ds(zbase, acc_stripe)])
            plsc.subcore_barrier()

            def body(j, carry):
                pltpu.sync_copy(ones_v, acc_sh.at[idx_v.at[j]], add=True)
                return carry

            lax.fori_loop(0, nch, body, 0)
            plsc.subcore_barrier()
            pltpu.sync_copy(acc_sh.at[pl.ds(base, out_stripe)],
                            out_hbm.at[c, pl.ds(base, out_stripe)])
            plsc.subcore_barrier()

        one_pass(srcr_v, osrc_hbm)
        one_pass(dstr_v, odst_hbm)

    return degree_kernel


def _make_agg_kernel(d, nch):
    out_stripe = HALF // NS
    acc_stripe = ACC_ROWS // NS

    @functools.partial(
        pl.kernel,
        out_type=jax.ShapeDtypeStruct((NC, HALF, d), jnp.float32),
        mesh=_sc_mesh(),
        scratch_types=[
            pltpu.VMEM((nch, CHUNK), jnp.int32),
            pltpu.VMEM((nch, CHUNK), jnp.int32),
            pltpu.VMEM((CHUNK, d), jnp.float32),
            pltpu.VMEM((CHUNK, d), jnp.float32),
            pltpu.VMEM_SHARED((ACC_ROWS, d), jnp.float32),
            pltpu.SemaphoreType.DMA,
            pltpu.SemaphoreType.DMA,
        ],
    )
    def agg_kernel(x_hbm, src_hbm, dstr_hbm, zeros_hbm, out_hbm,
                   src_v, dstr_v, rows0_v, rows1_v, acc_sh, sem0, sem1):
        c = lax.axis_index("c")
        s = lax.axis_index("s")
        wid = c * NS + s

        pltpu.sync_copy(src_hbm.at[s], src_v)
        pltpu.sync_copy(dstr_hbm.at[wid], dstr_v)

        zbase = s * acc_stripe
        pltpu.sync_copy(zeros_hbm.at[pl.ds(zbase, acc_stripe)],
                        acc_sh.at[pl.ds(zbase, acc_stripe)])

        plsc.subcore_barrier()

        def body(j, carry):
            pltpu.async_copy(x_hbm.at[src_v.at[j]], rows0_v, sem0).wait()
            pltpu.sync_copy(rows0_v, acc_sh.at[dstr_v.at[j]], add=True)
            return carry

        lax.fori_loop(0, nch, body, 0)

        plsc.subcore_barrier()

        base = s * out_stripe
        pltpu.sync_copy(acc_sh.at[pl.ds(base, out_stripe)],
                        out_hbm.at[c, pl.ds(base, out_stripe)])

    return agg_kernel


def _norm_body(dps_ref, dpd_ref, ns_ref, nd_ref):
    deg_s = dps_ref[:, 0:1]
    deg_d = dpd_ref[:, 0:1]
    ns_ref[...] = jnp.where(deg_s > 0, lax.rsqrt(jnp.maximum(deg_s, 1.0)), 0.0)
    nd_ref[...] = jnp.where(deg_d > 0, lax.rsqrt(jnp.maximum(deg_d, 1.0)), 0.0)


def _tc_norms(deg_src, deg_dst, block_rows):
    n = deg_src.shape[0]
    grid = n // block_rows
    return pl.pallas_call(
        _norm_body,
        grid=(grid,),
        in_specs=[
            pl.BlockSpec((block_rows, DW), lambda i: (i, 0)),
            pl.BlockSpec((block_rows, DW), lambda i: (i, 0)),
        ],
        out_specs=[
            pl.BlockSpec((block_rows, 1), lambda i: (i, 0)),
            pl.BlockSpec((block_rows, 1), lambda i: (i, 0)),
        ],
        out_shape=[
            jax.ShapeDtypeStruct((n, 1), jnp.float32),
            jax.ShapeDtypeStruct((n, 1), jnp.float32),
        ],
    )(deg_src, deg_dst)


def _scale_body(x_ref, n_ref, o_ref):
    o_ref[...] = x_ref[...] * n_ref[...]


def _tc_scale(x, nvec, block_rows):
    n, d = x.shape
    grid = n // block_rows
    return pl.pallas_call(
        _scale_body,
        grid=(grid,),
        in_specs=[
            pl.BlockSpec((block_rows, d), lambda i: (i, 0)),
            pl.BlockSpec((block_rows, 1), lambda i: (i, 0)),
        ],
        out_specs=pl.BlockSpec((block_rows, d), lambda i: (i, 0)),
        out_shape=jax.ShapeDtypeStruct((n, d), jnp.float32),
    )(x, nvec)


def _layer_body(p_ref, nd_ref, ps_ref, w_ref, b_ref, o_ref):
    agg = p_ref[...] * nd_ref[...]
    y = jnp.dot(agg, w_ref[...], preferred_element_type=jnp.float32) + b_ref[...]
    o_ref[...] = jnp.maximum(y, 0.0) * ps_ref[...]


def _tc_layer(p, norm_dst, post_scale, w, b, block_rows):
    n, d = p.shape
    grid = n // block_rows
    return pl.pallas_call(
        _layer_body,
        grid=(grid,),
        in_specs=[
            pl.BlockSpec((block_rows, d), lambda i: (i, 0)),
            pl.BlockSpec((block_rows, 1), lambda i: (i, 0)),
            pl.BlockSpec((block_rows, 1), lambda i: (i, 0)),
            pl.BlockSpec((d, d), lambda i: (0, 0)),
            pl.BlockSpec((1, d), lambda i: (0, 0)),
        ],
        out_specs=pl.BlockSpec((block_rows, d), lambda i: (i, 0)),
        out_shape=jax.ShapeDtypeStruct((n, d), jnp.float32),
    )(p, norm_dst, post_scale, w, b)


def kernel(features, edge_index, W1, b1, W2, b2):
    n, d = features.shape
    e = edge_index.shape[1]
    assert n <= NPAD and d % 128 == 0
    nch = -(-e // (NS * CHUNK))

    ep = NS * nch * CHUNK
    src_deg = jnp.pad(edge_index[0], (0, ep - e), constant_values=NPAD)
    dst = jnp.pad(edge_index[1], (0, ep - e), constant_values=NPAD)
    src2 = src_deg.reshape(ep // 128, 128)
    dst2 = dst.reshape(ep // 128, 128)
    srcr, dstr = _tc_redirect(src2, dst2)
    srcr_c = srcr.reshape(NC * NS, nch, CHUNK)
    dstr_c = dstr.reshape(NC * NS, nch, CHUNK)
    src_c = jnp.pad(edge_index[0], (0, ep - e)).reshape(NS, nch, CHUNK)

    zeros_deg = jnp.zeros((ACC_ROWS, DW), jnp.float32)
    zeros_agg = jnp.zeros((ACC_ROWS, d), jnp.float32)
    ones_deg = jnp.ones((CHUNK, DW), jnp.float32)

    block_rows = 2048
    assert NPAD % block_rows == 0
    degp_src, degp_dst = _make_degree_kernel(nch)(
        srcr_c, dstr_c, zeros_deg, ones_deg)
    norm_src, norm_dst = _tc_norms(
        degp_src.reshape(NPAD, DW), degp_dst.reshape(NPAD, DW), block_rows
    )

    features_p = jnp.pad(features, ((0, NPAD - n), (0, 0)))
    agg = _make_agg_kernel(d, nch)

    x0 = _tc_scale(features_p, norm_src, block_rows)
    p = agg(x0, src_c, dstr_c, zeros_agg)
    h1 = _tc_layer(p.reshape(NPAD, d), norm_dst, norm_src, W1,
                   b1.reshape(1, d), block_rows)
    p2 = agg(h1, src_c, dstr_c, zeros_agg)
    ones = jnp.ones((NPAD, 1), jnp.float32)
    h2 = _tc_layer(p2.reshape(NPAD, d), norm_dst, ones, W2,
                   b2.reshape(1, d), block_rows)
    return h2[:n]


# speedup vs baseline: 1.2651x; 1.1795x over previous
import functools

import jax
import jax.numpy as jnp
from jax import lax
from jax.experimental import pallas as pl
from jax.experimental.pallas import tpu as pltpu
from jax.experimental.pallas import tpu_sc as plsc

NC = 2
NS = 16
LANES = 16
CHUNK = 80
HALF = 5120
NPAD = NC * HALF
TRASH = 128
ACC_ROWS = HALF + TRASH
DW = 128


def _sc_mesh():
    return plsc.VectorSubcoreMesh(
        core_axis_name="c", subcore_axis_name="s", num_cores=NC, num_subcores=NS
    )


def _redirect_body(s_ref, d_ref, os_ref, od_ref):
    s = s_ref[...]
    d = d_ref[...]
    for c in range(NC):
        base = c * HALF
        sl = s - base
        dl = d - base
        os_ref[c] = jnp.where((sl >= 0) & (sl < HALF), sl, HALF)
        od_ref[c] = jnp.where((dl >= 0) & (dl < HALF), dl, HALF)


def _tc_redirect(src2, dst2):
    r, w = src2.shape
    oshape = jax.ShapeDtypeStruct((NC, r, w), jnp.int32)
    return pl.pallas_call(
        _redirect_body,
        out_shape=[oshape, oshape],
    )(src2, dst2)


def _make_degree_kernel(nch):
    out_stripe = HALF // NS
    acc_stripe = ACC_ROWS // NS

    @functools.partial(
        pl.kernel,
        out_type=[
            jax.ShapeDtypeStruct((NC, HALF, DW), jnp.float32),
            jax.ShapeDtypeStruct((NC, HALF, DW), jnp.float32),
        ],
        mesh=_sc_mesh(),
        scratch_types=[
            pltpu.VMEM((nch, CHUNK), jnp.int32),
            pltpu.VMEM((nch, CHUNK), jnp.int32),
            pltpu.VMEM((CHUNK, DW), jnp.float32),
            pltpu.VMEM_SHARED((ACC_ROWS, DW), jnp.float32),
        ],
    )
    def degree_kernel(srcr_hbm, dstr_hbm, zeros_hbm, ones_hbm, osrc_hbm,
                      odst_hbm, srcr_v, dstr_v, ones_v, acc_sh):
        c = lax.axis_index("c")
        s = lax.axis_index("s")
        wid = c * NS + s

        pltpu.sync_copy(srcr_hbm.at[wid], srcr_v)
        pltpu.sync_copy(dstr_hbm.at[wid], dstr_v)
        pltpu.sync_copy(ones_hbm, ones_v)

        zbase = s * acc_stripe
        base = s * out_stripe

        def one_pass(idx_v, out_hbm):
            pltpu.sync_copy(zeros_hbm.at[pl.ds(zbase, acc_stripe)],
                            acc_sh.at[pl.ds(zbase, acc_stripe)])
            plsc.subcore_barrier()

            def body(j, carry):
                pltpu.sync_copy(ones_v, acc_sh.at[idx_v.at[j]], add=True)
                return carry

            lax.fori_loop(0, nch, body, 0)
            plsc.subcore_barrier()
            pltpu.sync_copy(acc_sh.at[pl.ds(base, out_stripe)],
                            out_hbm.at[c, pl.ds(base, out_stripe)])
            plsc.subcore_barrier()

        one_pass(srcr_v, osrc_hbm)
        one_pass(dstr_v, odst_hbm)

    return degree_kernel


def _make_agg_kernel(d, nch):
    out_stripe = HALF // NS
    acc_stripe = ACC_ROWS // NS

    @functools.partial(
        pl.kernel,
        out_type=jax.ShapeDtypeStruct((NC, HALF, d), jnp.float32),
        mesh=_sc_mesh(),
        scratch_types=[
            pltpu.VMEM((nch, CHUNK), jnp.int32),
            pltpu.VMEM((nch, CHUNK), jnp.int32),
            pltpu.VMEM((CHUNK, d), jnp.float32),
            pltpu.VMEM((CHUNK, d), jnp.float32),
            pltpu.VMEM_SHARED((ACC_ROWS, d), jnp.float32),
            pltpu.SemaphoreType.DMA,
            pltpu.SemaphoreType.DMA,
            pltpu.SemaphoreType.DMA,
            pltpu.SemaphoreType.DMA,
        ],
    )
    def agg_kernel(x_hbm, src_hbm, dstr_hbm, zeros_hbm, out_hbm,
                   src_v, dstr_v, rows0_v, rows1_v, acc_sh,
                   sem0, sem1, sem2, sem3):
        c = lax.axis_index("c")
        s = lax.axis_index("s")
        wid = c * NS + s

        pltpu.sync_copy(src_hbm.at[s], src_v)
        pltpu.sync_copy(dstr_hbm.at[wid], dstr_v)

        zbase = s * acc_stripe
        pltpu.sync_copy(zeros_hbm.at[pl.ds(zbase, acc_stripe)],
                        acc_sh.at[pl.ds(zbase, acc_stripe)])

        plsc.subcore_barrier()

        rows = (rows0_v, rows1_v)
        gsem = (sem0, sem1)
        ssem = (sem2, sem3)

        for b in range(2):
            pltpu.async_copy(x_hbm.at[src_v.at[b]], rows[b], gsem[b]).wait()
            pltpu.async_copy(rows[b], acc_sh.at[dstr_v.at[b]], ssem[b],
                             add=True)

        def body(k, carry):
            for b in range(2):
                j = 2 * k + b
                pltpu.make_async_copy(
                    rows[b], acc_sh.at[dstr_v.at[j]], ssem[b]).wait()
                pltpu.async_copy(
                    x_hbm.at[src_v.at[j + 2]], rows[b], gsem[b]).wait()
                pltpu.async_copy(
                    rows[b], acc_sh.at[dstr_v.at[j + 2]], ssem[b], add=True)
            return carry

        lax.fori_loop(0, nch // 2 - 1, body, 0)

        for b in range(2):
            pltpu.make_async_copy(
                rows[b], acc_sh.at[dstr_v.at[b]], ssem[b]).wait()

        plsc.subcore_barrier()

        base = s * out_stripe
        pltpu.sync_copy(acc_sh.at[pl.ds(base, out_stripe)],
                        out_hbm.at[c, pl.ds(base, out_stripe)])

    return agg_kernel


def _norm_body(dps_ref, dpd_ref, ns_ref, nd_ref):
    deg_s = dps_ref[:, 0:1]
    deg_d = dpd_ref[:, 0:1]
    ns_ref[...] = jnp.where(deg_s > 0, lax.rsqrt(jnp.maximum(deg_s, 1.0)), 0.0)
    nd_ref[...] = jnp.where(deg_d > 0, lax.rsqrt(jnp.maximum(deg_d, 1.0)), 0.0)


def _tc_norms(deg_src, deg_dst, block_rows):
    n = deg_src.shape[0]
    grid = n // block_rows
    return pl.pallas_call(
        _norm_body,
        grid=(grid,),
        in_specs=[
            pl.BlockSpec((block_rows, DW), lambda i: (i, 0)),
            pl.BlockSpec((block_rows, DW), lambda i: (i, 0)),
        ],
        out_specs=[
            pl.BlockSpec((block_rows, 1), lambda i: (i, 0)),
            pl.BlockSpec((block_rows, 1), lambda i: (i, 0)),
        ],
        out_shape=[
            jax.ShapeDtypeStruct((n, 1), jnp.float32),
            jax.ShapeDtypeStruct((n, 1), jnp.float32),
        ],
    )(deg_src, deg_dst)


def _scale_body(x_ref, n_ref, o_ref):
    o_ref[...] = x_ref[...] * n_ref[...]


def _tc_scale(x, nvec, block_rows):
    n, d = x.shape
    grid = n // block_rows
    return pl.pallas_call(
        _scale_body,
        grid=(grid,),
        in_specs=[
            pl.BlockSpec((block_rows, d), lambda i: (i, 0)),
            pl.BlockSpec((block_rows, 1), lambda i: (i, 0)),
        ],
        out_specs=pl.BlockSpec((block_rows, d), lambda i: (i, 0)),
        out_shape=jax.ShapeDtypeStruct((n, d), jnp.float32),
    )(x, nvec)


def _layer_body(p_ref, nd_ref, ps_ref, w_ref, b_ref, o_ref):
    agg = p_ref[...] * nd_ref[...]
    y = jnp.dot(agg, w_ref[...], preferred_element_type=jnp.float32) + b_ref[...]
    o_ref[...] = jnp.maximum(y, 0.0) * ps_ref[...]


def _tc_layer(p, norm_dst, post_scale, w, b, block_rows):
    n, d = p.shape
    grid = n // block_rows
    return pl.pallas_call(
        _layer_body,
        grid=(grid,),
        in_specs=[
            pl.BlockSpec((block_rows, d), lambda i: (i, 0)),
            pl.BlockSpec((block_rows, 1), lambda i: (i, 0)),
            pl.BlockSpec((block_rows, 1), lambda i: (i, 0)),
            pl.BlockSpec((d, d), lambda i: (0, 0)),
            pl.BlockSpec((1, d), lambda i: (0, 0)),
        ],
        out_specs=pl.BlockSpec((block_rows, d), lambda i: (i, 0)),
        out_shape=jax.ShapeDtypeStruct((n, d), jnp.float32),
    )(p, norm_dst, post_scale, w, b)


def kernel(features, edge_index, W1, b1, W2, b2):
    n, d = features.shape
    e = edge_index.shape[1]
    assert n <= NPAD and d % 128 == 0
    nch = -(-e // (NS * CHUNK))

    ep = NS * nch * CHUNK
    src_deg = jnp.pad(edge_index[0], (0, ep - e), constant_values=NPAD)
    dst = jnp.pad(edge_index[1], (0, ep - e), constant_values=NPAD)
    src2 = src_deg.reshape(ep // 128, 128)
    dst2 = dst.reshape(ep // 128, 128)
    srcr, dstr = _tc_redirect(src2, dst2)
    srcr_c = srcr.reshape(NC * NS, nch, CHUNK)
    dstr_c = dstr.reshape(NC * NS, nch, CHUNK)
    src_c = jnp.pad(edge_index[0], (0, ep - e)).reshape(NS, nch, CHUNK)

    zeros_deg = jnp.zeros((ACC_ROWS, DW), jnp.float32)
    zeros_agg = jnp.zeros((ACC_ROWS, d), jnp.float32)
    ones_deg = jnp.ones((CHUNK, DW), jnp.float32)

    block_rows = 2048
    assert NPAD % block_rows == 0
    degp_src, degp_dst = _make_degree_kernel(nch)(
        srcr_c, dstr_c, zeros_deg, ones_deg)
    norm_src, norm_dst = _tc_norms(
        degp_src.reshape(NPAD, DW), degp_dst.reshape(NPAD, DW), block_rows
    )

    features_p = jnp.pad(features, ((0, NPAD - n), (0, 0)))
    agg = _make_agg_kernel(d, nch)

    x0 = _tc_scale(features_p, norm_src, block_rows)
    p = agg(x0, src_c, dstr_c, zeros_agg)
    h1 = _tc_layer(p.reshape(NPAD, d), norm_dst, norm_src, W1,
                   b1.reshape(1, d), block_rows)
    p2 = agg(h1, src_c, dstr_c, zeros_agg)
    ones = jnp.ones((NPAD, 1), jnp.float32)
    h2 = _tc_layer(p2.reshape(NPAD, d), norm_dst, ones, W2,
                   b2.reshape(1, d), block_rows)
    return h2[:n]


# speedup vs baseline: 1.2670x; 1.0016x over previous
import functools

import jax
import jax.numpy as jnp
from jax import lax
from jax.experimental import pallas as pl
from jax.experimental.pallas import tpu as pltpu
from jax.experimental.pallas import tpu_sc as plsc

NC = 2
NS = 16
LANES = 16
CHUNK = 80
HALF = 5120
NPAD = NC * HALF
TRASH = 128
ACC_ROWS = HALF + TRASH
DW = 128


def _sc_mesh():
    return plsc.VectorSubcoreMesh(
        core_axis_name="c", subcore_axis_name="s", num_cores=NC, num_subcores=NS
    )


def _redirect_body(s_ref, d_ref, os_ref, od_ref):
    s = s_ref[...]
    d = d_ref[...]
    for c in range(NC):
        base = c * HALF
        sl = s - base
        dl = d - base
        os_ref[c] = jnp.where((sl >= 0) & (sl < HALF), sl, HALF)
        od_ref[c] = jnp.where((dl >= 0) & (dl < HALF), dl, HALF)


def _tc_redirect(src2, dst2):
    r, w = src2.shape
    oshape = jax.ShapeDtypeStruct((NC, r, w), jnp.int32)
    return pl.pallas_call(
        _redirect_body,
        out_shape=[oshape, oshape],
    )(src2, dst2)


def _make_degree_kernel(nch):
    out_stripe = HALF // NS
    acc_stripe = ACC_ROWS // NS

    @functools.partial(
        pl.kernel,
        out_type=[
            jax.ShapeDtypeStruct((NC, HALF, DW), jnp.float32),
            jax.ShapeDtypeStruct((NC, HALF, DW), jnp.float32),
        ],
        mesh=_sc_mesh(),
        scratch_types=[
            pltpu.VMEM((nch, CHUNK), jnp.int32),
            pltpu.VMEM((nch, CHUNK), jnp.int32),
            pltpu.VMEM((CHUNK, DW), jnp.float32),
            pltpu.VMEM_SHARED((ACC_ROWS, DW), jnp.float32),
            pltpu.SemaphoreType.DMA,
        ],
    )
    def degree_kernel(srcr_hbm, dstr_hbm, zeros_hbm, ones_hbm, osrc_hbm,
                      odst_hbm, srcr_v, dstr_v, ones_v, acc_sh, sem):
        c = lax.axis_index("c")
        s = lax.axis_index("s")
        wid = c * NS + s

        pltpu.sync_copy(srcr_hbm.at[wid], srcr_v)
        pltpu.sync_copy(dstr_hbm.at[wid], dstr_v)
        pltpu.sync_copy(ones_hbm, ones_v)

        zbase = s * acc_stripe
        base = s * out_stripe

        def one_pass(idx_v, out_hbm):
            pltpu.sync_copy(zeros_hbm.at[pl.ds(zbase, acc_stripe)],
                            acc_sh.at[pl.ds(zbase, acc_stripe)])
            plsc.subcore_barrier()

            for b in range(2):
                pltpu.async_copy(ones_v, acc_sh.at[idx_v.at[b]], sem,
                                 add=True)

            def body(j, carry):
                pltpu.async_copy(ones_v, acc_sh.at[idx_v.at[j + 2]], sem,
                                 add=True)
                pltpu.make_async_copy(ones_v, acc_sh.at[idx_v.at[j]],
                                      sem).wait()
                return carry

            lax.fori_loop(0, nch - 2, body, 0)
            for b in range(2):
                pltpu.make_async_copy(ones_v, acc_sh.at[idx_v.at[b]],
                                      sem).wait()
            plsc.subcore_barrier()
            pltpu.sync_copy(acc_sh.at[pl.ds(base, out_stripe)],
                            out_hbm.at[c, pl.ds(base, out_stripe)])
            plsc.subcore_barrier()

        one_pass(srcr_v, osrc_hbm)
        one_pass(dstr_v, odst_hbm)

    return degree_kernel


def _make_agg_kernel(d, nch):
    out_stripe = HALF // NS
    acc_stripe = ACC_ROWS // NS

    @functools.partial(
        pl.kernel,
        out_type=jax.ShapeDtypeStruct((NC, HALF, d), jnp.float32),
        mesh=_sc_mesh(),
        scratch_types=[
            pltpu.VMEM((nch, CHUNK), jnp.int32),
            pltpu.VMEM((nch, CHUNK), jnp.int32),
            pltpu.VMEM((CHUNK, d), jnp.float32),
            pltpu.VMEM((CHUNK, d), jnp.float32),
            pltpu.VMEM_SHARED((ACC_ROWS, d), jnp.float32),
            pltpu.SemaphoreType.DMA,
            pltpu.SemaphoreType.DMA,
            pltpu.SemaphoreType.DMA,
            pltpu.SemaphoreType.DMA,
        ],
    )
    def agg_kernel(x_hbm, src_hbm, dstr_hbm, zeros_hbm, out_hbm,
                   src_v, dstr_v, rows0_v, rows1_v, acc_sh,
                   sem0, sem1, sem2, sem3):
        c = lax.axis_index("c")
        s = lax.axis_index("s")
        wid = c * NS + s

        pltpu.sync_copy(src_hbm.at[s], src_v)
        pltpu.sync_copy(dstr_hbm.at[wid], dstr_v)

        zbase = s * acc_stripe
        pltpu.sync_copy(zeros_hbm.at[pl.ds(zbase, acc_stripe)],
                        acc_sh.at[pl.ds(zbase, acc_stripe)])

        plsc.subcore_barrier()

        rows = (rows0_v, rows1_v)
        gsem = (sem0, sem1)
        ssem = (sem2, sem3)

        for b in range(2):
            pltpu.async_copy(x_hbm.at[src_v.at[b]], rows[b], gsem[b]).wait()
            pltpu.async_copy(rows[b], acc_sh.at[dstr_v.at[b]], ssem[b],
                             add=True)

        def body(k, carry):
            for b in range(2):
                j = 2 * k + b
                pltpu.make_async_copy(
                    rows[b], acc_sh.at[dstr_v.at[j]], ssem[b]).wait()
                pltpu.async_copy(
                    x_hbm.at[src_v.at[j + 2]], rows[b], gsem[b]).wait()
                pltpu.async_copy(
                    rows[b], acc_sh.at[dstr_v.at[j + 2]], ssem[b], add=True)
            return carry

        lax.fori_loop(0, nch // 2 - 1, body, 0)

        for b in range(2):
            pltpu.make_async_copy(
                rows[b], acc_sh.at[dstr_v.at[b]], ssem[b]).wait()

        plsc.subcore_barrier()

        base = s * out_stripe
        pltpu.sync_copy(acc_sh.at[pl.ds(base, out_stripe)],
                        out_hbm.at[c, pl.ds(base, out_stripe)])

    return agg_kernel


def _norm_body(dps_ref, dpd_ref, ns_ref, nd_ref):
    deg_s = dps_ref[:, 0:1]
    deg_d = dpd_ref[:, 0:1]
    ns_ref[...] = jnp.where(deg_s > 0, lax.rsqrt(jnp.maximum(deg_s, 1.0)), 0.0)
    nd_ref[...] = jnp.where(deg_d > 0, lax.rsqrt(jnp.maximum(deg_d, 1.0)), 0.0)


def _tc_norms(deg_src, deg_dst, block_rows):
    n = deg_src.shape[0]
    grid = n // block_rows
    return pl.pallas_call(
        _norm_body,
        grid=(grid,),
        in_specs=[
            pl.BlockSpec((block_rows, DW), lambda i: (i, 0)),
            pl.BlockSpec((block_rows, DW), lambda i: (i, 0)),
        ],
        out_specs=[
            pl.BlockSpec((block_rows, 1), lambda i: (i, 0)),
            pl.BlockSpec((block_rows, 1), lambda i: (i, 0)),
        ],
        out_shape=[
            jax.ShapeDtypeStruct((n, 1), jnp.float32),
            jax.ShapeDtypeStruct((n, 1), jnp.float32),
        ],
    )(deg_src, deg_dst)


def _scale_body(x_ref, n_ref, o_ref):
    o_ref[...] = x_ref[...] * n_ref[...]


def _tc_scale(x, nvec, block_rows):
    n, d = x.shape
    grid = n // block_rows
    return pl.pallas_call(
        _scale_body,
        grid=(grid,),
        in_specs=[
            pl.BlockSpec((block_rows, d), lambda i: (i, 0)),
            pl.BlockSpec((block_rows, 1), lambda i: (i, 0)),
        ],
        out_specs=pl.BlockSpec((block_rows, d), lambda i: (i, 0)),
        out_shape=jax.ShapeDtypeStruct((n, d), jnp.float32),
    )(x, nvec)


def _layer_body(p_ref, nd_ref, ps_ref, w_ref, b_ref, o_ref):
    agg = p_ref[...] * nd_ref[...]
    y = jnp.dot(agg, w_ref[...], preferred_element_type=jnp.float32) + b_ref[...]
    o_ref[...] = jnp.maximum(y, 0.0) * ps_ref[...]


def _tc_layer(p, norm_dst, post_scale, w, b, block_rows):
    n, d = p.shape
    grid = n // block_rows
    return pl.pallas_call(
        _layer_body,
        grid=(grid,),
        in_specs=[
            pl.BlockSpec((block_rows, d), lambda i: (i, 0)),
            pl.BlockSpec((block_rows, 1), lambda i: (i, 0)),
            pl.BlockSpec((block_rows, 1), lambda i: (i, 0)),
            pl.BlockSpec((d, d), lambda i: (0, 0)),
            pl.BlockSpec((1, d), lambda i: (0, 0)),
        ],
        out_specs=pl.BlockSpec((block_rows, d), lambda i: (i, 0)),
        out_shape=jax.ShapeDtypeStruct((n, d), jnp.float32),
    )(p, norm_dst, post_scale, w, b)


def kernel(features, edge_index, W1, b1, W2, b2):
    n, d = features.shape
    e = edge_index.shape[1]
    assert n <= NPAD and d % 128 == 0
    nch = -(-e // (NS * CHUNK))

    ep = NS * nch * CHUNK
    src_deg = jnp.pad(edge_index[0], (0, ep - e), constant_values=NPAD)
    dst = jnp.pad(edge_index[1], (0, ep - e), constant_values=NPAD)
    src2 = src_deg.reshape(ep // 128, 128)
    dst2 = dst.reshape(ep // 128, 128)
    srcr, dstr = _tc_redirect(src2, dst2)
    srcr_c = srcr.reshape(NC * NS, nch, CHUNK)
    dstr_c = dstr.reshape(NC * NS, nch, CHUNK)
    src_c = jnp.pad(edge_index[0], (0, ep - e)).reshape(NS, nch, CHUNK)

    zeros_deg = jnp.zeros((ACC_ROWS, DW), jnp.float32)
    zeros_agg = jnp.zeros((ACC_ROWS, d), jnp.float32)
    ones_deg = jnp.ones((CHUNK, DW), jnp.float32)

    block_rows = 2048
    assert NPAD % block_rows == 0
    degp_src, degp_dst = _make_degree_kernel(nch)(
        srcr_c, dstr_c, zeros_deg, ones_deg)
    norm_src, norm_dst = _tc_norms(
        degp_src.reshape(NPAD, DW), degp_dst.reshape(NPAD, DW), block_rows
    )

    features_p = jnp.pad(features, ((0, NPAD - n), (0, 0)))
    agg = _make_agg_kernel(d, nch)

    x0 = _tc_scale(features_p, norm_src, block_rows)
    p = agg(x0, src_c, dstr_c, zeros_agg)
    h1 = _tc_layer(p.reshape(NPAD, d), norm_dst, norm_src, W1,
                   b1.reshape(1, d), block_rows)
    p2 = agg(h1, src_c, dstr_c, zeros_agg)
    ones = jnp.ones((NPAD, 1), jnp.float32)
    h2 = _tc_layer(p2.reshape(NPAD, d), norm_dst, ones, W2,
                   b2.reshape(1, d), block_rows)
    return h2[:n]
